# baseline (speedup 1.0000x reference)
import functools

import jax
import jax.numpy as jnp
from jax import lax
from jax.experimental import pallas as pl
from jax.experimental.pallas import tpu as pltpu
from jax.experimental.pallas import tpu_sc as plsc

_VOCAB = 100000
_DIM = 64
_BATCH = 1024
_TV = 2048


def _gather_rows_sc(emb, idx):
    info = plsc.get_sparse_core_info()
    nc, ns = info.num_cores, info.num_subcores
    nw = nc * ns
    bpw = _BATCH // nw
    mesh = plsc.VectorSubcoreMesh(core_axis_name="c", subcore_axis_name="s")

    @functools.partial(
        pl.kernel,
        mesh=mesh,
        compiler_params=pltpu.CompilerParams(use_tc_tiling_on_sc=False),
        out_type=jax.ShapeDtypeStruct((_BATCH, _DIM), jnp.float32),
        scratch_types=[
            pltpu.VMEM((bpw,), jnp.int32),
            pltpu.VMEM((bpw, _DIM), jnp.float32),
            pltpu.SemaphoreType.DMA,
        ],
    )
    def gk(emb_hbm, idx_hbm, out_hbm, idx_v, rows_v, sem):
        wid = lax.axis_index("s") * nc + lax.axis_index("c")
        base = wid * bpw
        pltpu.sync_copy(idx_hbm.at[pl.ds(base, bpw)], idx_v)
        pltpu.async_copy(emb_hbm.at[idx_v], rows_v, sem).wait()
        pltpu.sync_copy(rows_v, out_hbm.at[pl.ds(base, bpw)])

    return gk(emb, idx)


def _matmul_body(x_ref, w_ref, b_ref, out_ref):
    xb = x_ref[...].astype(jnp.bfloat16)
    wb = w_ref[...].astype(jnp.bfloat16)
    acc = jnp.dot(xb, wb, preferred_element_type=jnp.float32)
    out_ref[...] = acc + b_ref[...]


def kernel(input_ids, emb, W, b):
    idx = input_ids.astype(jnp.int32)
    x = _gather_rows_sc(emb, idx)
    b2 = b.reshape(1, _VOCAB)
    logits = pl.pallas_call(
        _matmul_body,
        grid=(pl.cdiv(_VOCAB, _TV),),
        in_specs=[
            pl.BlockSpec((_BATCH, _DIM), lambda i: (0, 0)),
            pl.BlockSpec((_DIM, _TV), lambda i: (0, i)),
            pl.BlockSpec((1, _TV), lambda i: (0, i)),
        ],
        out_specs=pl.BlockSpec((_BATCH, _TV), lambda i: (0, i)),
        out_shape=jax.ShapeDtypeStruct((_BATCH, _VOCAB), jnp.float32),
        compiler_params=pltpu.CompilerParams(
            dimension_semantics=("arbitrary",),
        ),
    )(x, W, b2)
    return logits


# speedup vs baseline: 2.6288x; 2.6288x over previous
import functools

import jax
import jax.numpy as jnp
from jax import lax
from jax.experimental import pallas as pl
from jax.experimental.pallas import tpu as pltpu
from jax.experimental.pallas import tpu_sc as plsc

_VOCAB = 100000
_DIM = 64
_BATCH = 1024
_TV = 2048


def _gather_rows_sc(emb, idx):
    info = plsc.get_sparse_core_info()
    nc, ns = info.num_cores, info.num_subcores
    nw = nc * ns
    bpw = _BATCH // nw
    mesh = plsc.VectorSubcoreMesh(core_axis_name="c", subcore_axis_name="s")

    @functools.partial(
        pl.kernel,
        mesh=mesh,
        compiler_params=pltpu.CompilerParams(use_tc_tiling_on_sc=False),
        out_type=jax.ShapeDtypeStruct((_BATCH, _DIM), jnp.float32),
        scratch_types=[
            pltpu.VMEM((bpw,), jnp.int32),
            pltpu.VMEM((bpw, _DIM), jnp.float32),
            pltpu.SemaphoreType.DMA,
        ],
    )
    def gk(emb_hbm, idx_hbm, out_hbm, idx_v, rows_v, sem):
        wid = lax.axis_index("s") * nc + lax.axis_index("c")
        base = wid * bpw
        pltpu.sync_copy(idx_hbm.at[pl.ds(base, bpw)], idx_v)
        pltpu.async_copy(emb_hbm.at[idx_v], rows_v, sem).wait()
        pltpu.sync_copy(rows_v, out_hbm.at[pl.ds(base, bpw)])

    return gk(emb, idx)


def _matmul_body(xt_ref, w_ref, b_ref, out_ref):
    lhs = jnp.concatenate([w_ref[...], b_ref[...]], axis=0).astype(jnp.bfloat16)
    rhs = jnp.concatenate(
        [xt_ref[...], jnp.ones((1, _BATCH), jnp.float32)], axis=0
    ).astype(jnp.bfloat16)
    out_ref[...] = jax.lax.dot_general(
        lhs, rhs, (((0,), (0,)), ((), ())), preferred_element_type=jnp.float32
    )


def kernel(input_ids, emb, W, b):
    idx = input_ids.astype(jnp.int32)
    x = _gather_rows_sc(emb, idx)
    xt = jnp.swapaxes(x, 0, 1)
    b2 = b.reshape(1, _VOCAB)
    logits_t = pl.pallas_call(
        _matmul_body,
        grid=(pl.cdiv(_VOCAB, _TV),),
        in_specs=[
            pl.BlockSpec((_DIM, _BATCH), lambda i: (0, 0)),
            pl.BlockSpec((_DIM, _TV), lambda i: (0, i)),
            pl.BlockSpec((1, _TV), lambda i: (0, i)),
        ],
        out_specs=pl.BlockSpec((_TV, _BATCH), lambda i: (i, 0)),
        out_shape=jax.ShapeDtypeStruct((_VOCAB, _BATCH), jnp.float32),
        compiler_params=pltpu.CompilerParams(
            dimension_semantics=("arbitrary",),
        ),
    )(xt, W, b2)
    return jnp.transpose(logits_t)


# speedup vs baseline: 3.0158x; 1.1472x over previous
import functools

import jax
import jax.numpy as jnp
from jax import lax
from jax.experimental import pallas as pl
from jax.experimental.pallas import tpu as pltpu
from jax.experimental.pallas import tpu_sc as plsc

_VOCAB = 100000
_DIM = 64
_BATCH = 1024
_TV = 2048


def _gather_rows_sc(emb, idx):
    info = plsc.get_sparse_core_info()
    nc, ns, nl = info.num_cores, info.num_subcores, info.num_lanes
    nw = nc * ns
    bpw = _BATCH // nw
    mesh = plsc.VectorSubcoreMesh(core_axis_name="c", subcore_axis_name="s")

    @functools.partial(
        pl.kernel,
        mesh=mesh,
        compiler_params=pltpu.CompilerParams(needs_layout_passes=False),
        out_type=jax.ShapeDtypeStruct((_BATCH, _DIM), jnp.float32),
        scratch_types=[
            pltpu.VMEM((bpw,), jnp.int32),
            pltpu.VMEM((bpw, _DIM), jnp.float32),
            pltpu.SemaphoreType.DMA,
            pltpu.SemaphoreType.DMA,
        ],
    )
    def gk(emb_hbm, idx_hbm, out_hbm, idx_v, rows_v, isem, rsem):
        wid = lax.axis_index("s") * nc + lax.axis_index("c")
        base = wid * bpw
        pltpu.sync_copy(idx_hbm.at[pl.ds(base, bpw)], idx_v)
        copies = []
        for i in range(bpw):
            group = idx_v[pl.ds(i - i % nl, nl)]
            sel = lax.iota(jnp.int32, nl) == (i % nl)
            row = lax.reduce_max(jnp.where(sel, group, 0), (0,))
            copies.append(
                pltpu.async_copy(
                    emb_hbm.at[pl.ds(row, 1), :], rows_v.at[pl.ds(i, 1), :], rsem
                )
            )
        for c in copies:
            c.wait()
        pltpu.sync_copy(rows_v, out_hbm.at[pl.ds(base, bpw)])

    return gk(emb, idx)


def _matmul_body(xt_ref, w_ref, b_ref, out_ref):
    lhs = jnp.concatenate([w_ref[...], b_ref[...]], axis=0).astype(jnp.bfloat16)
    rhs = jnp.concatenate(
        [xt_ref[...], jnp.ones((1, _BATCH), jnp.float32)], axis=0
    ).astype(jnp.bfloat16)
    out_ref[...] = jax.lax.dot_general(
        lhs, rhs, (((0,), (0,)), ((), ())), preferred_element_type=jnp.float32
    )


def kernel(input_ids, emb, W, b):
    idx = input_ids.astype(jnp.int32)
    x = _gather_rows_sc(emb, idx)
    xt = jnp.swapaxes(x, 0, 1)
    b2 = b.reshape(1, _VOCAB)
    logits_t = pl.pallas_call(
        _matmul_body,
        grid=(pl.cdiv(_VOCAB, _TV),),
        in_specs=[
            pl.BlockSpec((_DIM, _BATCH), lambda i: (0, 0)),
            pl.BlockSpec((_DIM, _TV), lambda i: (0, i)),
            pl.BlockSpec((1, _TV), lambda i: (0, i)),
        ],
        out_specs=pl.BlockSpec((_TV, _BATCH), lambda i: (i, 0)),
        out_shape=jax.ShapeDtypeStruct((_VOCAB, _BATCH), jnp.float32),
        compiler_params=pltpu.CompilerParams(
            dimension_semantics=("arbitrary",),
        ),
    )(xt, W, b2)
    return jnp.transpose(logits_t)


# speedup vs baseline: 3.0650x; 1.0163x over previous
import functools

import jax
import jax.numpy as jnp
from jax import lax
from jax.experimental import pallas as pl
from jax.experimental.pallas import tpu as pltpu

_VOCAB = 100000
_DIM = 64
_BATCH = 1024
_TV = 2048


def _fused_body(idx_ref, emb_any, w_ref, b_ref, out_ref, x_v, xt_v, sem):
    @pl.when(pl.program_id(0) == 0)
    def _gather():
        def _issue(i, _):
            pltpu.async_copy(
                emb_any.at[pl.ds(idx_ref[i], 1), :], x_v.at[pl.ds(i, 1), :], sem
            )
            return _

        lax.fori_loop(0, _BATCH, _issue, 0)

        def _drain(i, _):
            pltpu.make_async_copy(
                emb_any.at[pl.ds(idx_ref[i], 1), :], x_v.at[pl.ds(i, 1), :], sem
            ).wait()
            return _

        lax.fori_loop(0, _BATCH, _drain, 0)
        xt_v[...] = x_v[...].T

    lhs = jnp.concatenate([w_ref[...], b_ref[...]], axis=0).astype(jnp.bfloat16)
    rhs = jnp.concatenate(
        [xt_v[...], jnp.ones((1, _BATCH), jnp.float32)], axis=0
    ).astype(jnp.bfloat16)
    out_ref[...] = jax.lax.dot_general(
        lhs, rhs, (((0,), (0,)), ((), ())), preferred_element_type=jnp.float32
    )


def kernel(input_ids, emb, W, b):
    idx = input_ids.astype(jnp.int32)
    b2 = b.reshape(1, _VOCAB)
    logits_t = pl.pallas_call(
        _fused_body,
        grid_spec=pltpu.PrefetchScalarGridSpec(
            num_scalar_prefetch=1,
            grid=(pl.cdiv(_VOCAB, _TV),),
            in_specs=[
                pl.BlockSpec(memory_space=pl.ANY),
                pl.BlockSpec((_DIM, _TV), lambda i, *_: (0, i)),
                pl.BlockSpec((1, _TV), lambda i, *_: (0, i)),
            ],
            out_specs=pl.BlockSpec((_TV, _BATCH), lambda i, *_: (i, 0)),
            scratch_shapes=[
                pltpu.VMEM((_BATCH, _DIM), jnp.float32),
                pltpu.VMEM((_DIM, _BATCH), jnp.float32),
                pltpu.SemaphoreType.DMA,
            ],
        ),
        out_shape=jax.ShapeDtypeStruct((_VOCAB, _BATCH), jnp.float32),
        compiler_params=pltpu.CompilerParams(
            dimension_semantics=("arbitrary",),
        ),
    )(idx, emb, W, b2)
    return jnp.transpose(logits_t)


# speedup vs baseline: 3.3633x; 1.0973x over previous
import functools

import jax
import jax.numpy as jnp
from jax import lax
from jax.experimental import pallas as pl
from jax.experimental.pallas import tpu as pltpu
from jax.experimental.pallas import tpu_sc as plsc

_VOCAB = 100000
_DIM = 64
_BATCH = 1024
_TV = 2048
_NB = 8


def _gather_rows_sc(embt, idx):
    info = plsc.get_sparse_core_info()
    nc, ns, nl = info.num_cores, info.num_subcores, info.num_lanes
    nw = nc * ns
    bpw = _BATCH // nw
    mesh = plsc.VectorSubcoreMesh(core_axis_name="c", subcore_axis_name="s")

    @functools.partial(
        pl.kernel,
        mesh=mesh,
        compiler_params=pltpu.CompilerParams(needs_layout_passes=False),
        out_type=jax.ShapeDtypeStruct((_BATCH, _DIM), jnp.float32),
        scratch_types=[
            pltpu.VMEM((bpw,), jnp.int32),
            pltpu.VMEM((_DIM, _NB * 128), jnp.float32),
            pltpu.VMEM((bpw, _DIM), jnp.float32),
            pltpu.SemaphoreType.DMA((_NB,)),
        ],
    )
    def gk(embt_hbm, idx_hbm, out_hbm, idx_v, blk_v, rows_v, sems):
        wid = lax.axis_index("s") * nc + lax.axis_index("c")
        base = wid * bpw
        pltpu.sync_copy(idx_hbm.at[pl.ds(base, bpw)], idx_v)

        def token_col(i):
            group = idx_v[pl.ds(i - i % nl, nl)]
            sel = lax.iota(jnp.int32, nl) == (i % nl)
            return lax.reduce_max(jnp.where(sel, group, 0), (0,))

        def fire(i):
            col = token_col(i)
            col0 = pl.multiple_of((col // 128) * 128, 128)
            s = i % _NB
            return (
                pltpu.async_copy(
                    embt_hbm.at[:, pl.ds(col0, 128)],
                    blk_v.at[:, pl.ds(s * 128, 128)],
                    sems.at[s],
                ),
                col - col0,
            )

        ring = [fire(i) for i in range(_NB)]
        for i in range(bpw):
            desc, r = ring[i % _NB]
            desc.wait()
            lane = i % _NB * 128 + r
            for k in range(_DIM // nl):
                d = lax.iota(jnp.int32, nl) + k * nl
                vals = plsc.load_gather(blk_v, [d, jnp.full((nl,), 0, jnp.int32) + lane])
                plsc.store_scatter(
                    rows_v, [jnp.full((nl,), i, jnp.int32), d], vals
                )
            if i + _NB < bpw:
                ring[i % _NB] = fire(i + _NB)
        pltpu.sync_copy(rows_v, out_hbm.at[pl.ds(base, bpw)])

    return gk(embt, idx)


def _matmul_body(xt_ref, w_ref, b_ref, out_ref):
    lhs = jnp.concatenate([w_ref[...], b_ref[...]], axis=0).astype(jnp.bfloat16)
    rhs = jnp.concatenate(
        [xt_ref[...], jnp.ones((1, _BATCH), jnp.float32)], axis=0
    ).astype(jnp.bfloat16)
    out_ref[...] = jax.lax.dot_general(
        lhs, rhs, (((0,), (0,)), ((), ())), preferred_element_type=jnp.float32
    )


def kernel(input_ids, emb, W, b):
    idx = input_ids.astype(jnp.int32)
    embt = jnp.swapaxes(emb, 0, 1)
    x = _gather_rows_sc(embt, idx)
    xt = jnp.swapaxes(x, 0, 1)
    b2 = b.reshape(1, _VOCAB)
    logits_t = pl.pallas_call(
        _matmul_body,
        grid=(pl.cdiv(_VOCAB, _TV),),
        in_specs=[
            pl.BlockSpec((_DIM, _BATCH), lambda i: (0, 0)),
            pl.BlockSpec((_DIM, _TV), lambda i: (0, i)),
            pl.BlockSpec((1, _TV), lambda i: (0, i)),
        ],
        out_specs=pl.BlockSpec((_TV, _BATCH), lambda i: (i, 0)),
        out_shape=jax.ShapeDtypeStruct((_VOCAB, _BATCH), jnp.float32),
        compiler_params=pltpu.CompilerParams(
            dimension_semantics=("arbitrary",),
        ),
    )(xt, W, b2)
    return jnp.transpose(logits_t)


# speedup vs baseline: 3.3698x; 1.0019x over previous
import functools

import jax
import jax.numpy as jnp
from jax import lax
from jax.experimental import pallas as pl
from jax.experimental.pallas import tpu as pltpu
from jax.experimental.pallas import tpu_sc as plsc

_VOCAB = 100000
_DIM = 64
_BATCH = 1024
_TV = 2048
_NB = 8


def _gather_rows_sc(embt, idx):
    info = plsc.get_sparse_core_info()
    nc, ns, nl = info.num_cores, info.num_subcores, info.num_lanes
    nw = nc * ns
    bpw = _BATCH // nw
    mesh = plsc.VectorSubcoreMesh(core_axis_name="c", subcore_axis_name="s")

    @functools.partial(
        pl.kernel,
        mesh=mesh,
        compiler_params=pltpu.CompilerParams(needs_layout_passes=False),
        out_type=jax.ShapeDtypeStruct((_BATCH, _DIM), jnp.float32),
        scratch_types=[
            pltpu.VMEM((bpw,), jnp.int32),
            pltpu.VMEM((_DIM, _NB * 128), jnp.float32),
            pltpu.VMEM((bpw, _DIM), jnp.float32),
            pltpu.SemaphoreType.DMA((_NB,)),
        ],
    )
    def gk(embt_hbm, idx_hbm, out_hbm, idx_v, blk_v, rows_v, sems):
        wid = lax.axis_index("s") * nc + lax.axis_index("c")
        base = wid * bpw
        pltpu.sync_copy(idx_hbm.at[pl.ds(base, bpw)], idx_v)

        def token_col(i):
            group = idx_v[pl.ds(i - i % nl, nl)]
            sel = lax.iota(jnp.int32, nl) == (i % nl)
            return lax.reduce_max(jnp.where(sel, group, 0), (0,))

        def fire(i):
            col = token_col(i)
            col0 = pl.multiple_of((col // 128) * 128, 128)
            s = i % _NB
            return (
                pltpu.async_copy(
                    embt_hbm.at[:, pl.ds(col0, 128)],
                    blk_v.at[:, pl.ds(s * 128, 128)],
                    sems.at[s],
                ),
                col - col0,
            )

        ring = [fire(i) for i in range(_NB)]
        for i in range(bpw):
            desc, r = ring[i % _NB]
            desc.wait()
            lane = i % _NB * 128 + r
            for k in range(_DIM // nl):
                d = lax.iota(jnp.int32, nl) + k * nl
                vals = plsc.load_gather(blk_v, [d, jnp.full((nl,), 0, jnp.int32) + lane])
                plsc.store_scatter(
                    rows_v, [jnp.full((nl,), i, jnp.int32), d], vals
                )
            if i + _NB < bpw:
                ring[i % _NB] = fire(i + _NB)
        pltpu.sync_copy(rows_v, out_hbm.at[pl.ds(base, bpw)])

    return gk(embt, idx)


def _matmul_body(x_ref, w_ref, b_ref, out_ref):
    lhs = jnp.concatenate([w_ref[...], b_ref[...]], axis=0).astype(jnp.bfloat16)
    rhs = jnp.concatenate(
        [x_ref[...], jnp.ones((_BATCH, 1), jnp.float32)], axis=1
    ).astype(jnp.bfloat16)
    out_ref[...] = jax.lax.dot_general(
        lhs, rhs, (((0,), (1,)), ((), ())), preferred_element_type=jnp.float32
    )


def kernel(input_ids, emb, W, b):
    idx = input_ids.astype(jnp.int32)
    embt = jnp.swapaxes(emb, 0, 1)
    x = _gather_rows_sc(embt, idx)
    b2 = b.reshape(1, _VOCAB)
    logits_t = pl.pallas_call(
        _matmul_body,
        grid=(pl.cdiv(_VOCAB, _TV),),
        in_specs=[
            pl.BlockSpec((_BATCH, _DIM), lambda i: (0, 0)),
            pl.BlockSpec((_DIM, _TV), lambda i: (0, i)),
            pl.BlockSpec((1, _TV), lambda i: (0, i)),
        ],
        out_specs=pl.BlockSpec((_TV, _BATCH), lambda i: (i, 0)),
        out_shape=jax.ShapeDtypeStruct((_VOCAB, _BATCH), jnp.float32),
        compiler_params=pltpu.CompilerParams(
            dimension_semantics=("arbitrary",),
        ),
    )(x, W, b2)
    return jnp.transpose(logits_t)


# speedup vs baseline: 3.3983x; 1.0084x over previous
import functools

import jax
import jax.numpy as jnp
from jax import lax
from jax.experimental import pallas as pl
from jax.experimental.pallas import tpu as pltpu
from jax.experimental.pallas import tpu_sc as plsc

_VOCAB = 100000
_DIM = 64
_BATCH = 1024
_TV = 4096
_NB = 8


def _gather_rows_sc(embt, idx):
    info = plsc.get_sparse_core_info()
    nc, ns, nl = info.num_cores, info.num_subcores, info.num_lanes
    nw = nc * ns
    bpw = _BATCH // nw
    mesh = plsc.VectorSubcoreMesh(core_axis_name="c", subcore_axis_name="s")

    @functools.partial(
        pl.kernel,
        mesh=mesh,
        compiler_params=pltpu.CompilerParams(needs_layout_passes=False),
        out_type=jax.ShapeDtypeStruct((_BATCH, _DIM), jnp.float32),
        scratch_types=[
            pltpu.VMEM((bpw,), jnp.int32),
            pltpu.VMEM((_DIM, _NB * 128), jnp.float32),
            pltpu.VMEM((bpw, _DIM), jnp.float32),
            pltpu.SemaphoreType.DMA((_NB,)),
        ],
    )
    def gk(embt_hbm, idx_hbm, out_hbm, idx_v, blk_v, rows_v, sems):
        wid = lax.axis_index("s") * nc + lax.axis_index("c")
        base = wid * bpw
        pltpu.sync_copy(idx_hbm.at[pl.ds(base, bpw)], idx_v)

        def token_col(i):
            group = idx_v[pl.ds(i - i % nl, nl)]
            sel = lax.iota(jnp.int32, nl) == (i % nl)
            return lax.reduce_max(jnp.where(sel, group, 0), (0,))

        def fire(i):
            col = token_col(i)
            col0 = pl.multiple_of((col // 128) * 128, 128)
            s = i % _NB
            return (
                pltpu.async_copy(
                    embt_hbm.at[:, pl.ds(col0, 128)],
                    blk_v.at[:, pl.ds(s * 128, 128)],
                    sems.at[s],
                ),
                col - col0,
            )

        ring = [fire(i) for i in range(_NB)]
        for i in range(bpw):
            desc, r = ring[i % _NB]
            desc.wait()
            lane = i % _NB * 128 + r
            for k in range(_DIM // nl):
                d = lax.iota(jnp.int32, nl) + k * nl
                vals = plsc.load_gather(blk_v, [d, jnp.full((nl,), 0, jnp.int32) + lane])
                plsc.store_scatter(
                    rows_v, [jnp.full((nl,), i, jnp.int32), d], vals
                )
            if i + _NB < bpw:
                ring[i % _NB] = fire(i + _NB)
        pltpu.sync_copy(rows_v, out_hbm.at[pl.ds(base, bpw)])

    return gk(embt, idx)


def _matmul_body(x_ref, w_ref, b_ref, out_ref):
    lhs = jnp.concatenate([w_ref[...], b_ref[...]], axis=0).astype(jnp.bfloat16)
    rhs = jnp.concatenate(
        [x_ref[...], jnp.ones((_BATCH, 1), jnp.float32)], axis=1
    ).astype(jnp.bfloat16)
    out_ref[...] = jax.lax.dot_general(
        lhs, rhs, (((0,), (1,)), ((), ())), preferred_element_type=jnp.float32
    )


def kernel(input_ids, emb, W, b):
    idx = input_ids.astype(jnp.int32)
    embt = jnp.swapaxes(emb, 0, 1)
    x = _gather_rows_sc(embt, idx)
    b2 = b.reshape(1, _VOCAB)
    logits_t = pl.pallas_call(
        _matmul_body,
        grid=(pl.cdiv(_VOCAB, _TV),),
        in_specs=[
            pl.BlockSpec((_BATCH, _DIM), lambda i: (0, 0)),
            pl.BlockSpec((_DIM, _TV), lambda i: (0, i)),
            pl.BlockSpec((1, _TV), lambda i: (0, i)),
        ],
        out_specs=pl.BlockSpec((_TV, _BATCH), lambda i: (i, 0)),
        out_shape=jax.ShapeDtypeStruct((_VOCAB, _BATCH), jnp.float32),
        compiler_params=pltpu.CompilerParams(
            dimension_semantics=("arbitrary",),
        ),
    )(x, W, b2)
    return jnp.transpose(logits_t)


# speedup vs baseline: 3.4042x; 1.0017x over previous
import functools

import jax
import jax.numpy as jnp
from jax import lax
from jax.experimental import pallas as pl
from jax.experimental.pallas import tpu as pltpu
from jax.experimental.pallas import tpu_sc as plsc

_VOCAB = 100000
_DIM = 64
_BATCH = 1024
_TV = 6144
_NB = 8


def _gather_rows_sc(embt, idx):
    info = plsc.get_sparse_core_info()
    nc, ns, nl = info.num_cores, info.num_subcores, info.num_lanes
    nw = nc * ns
    bpw = _BATCH // nw
    mesh = plsc.VectorSubcoreMesh(core_axis_name="c", subcore_axis_name="s")

    @functools.partial(
        pl.kernel,
        mesh=mesh,
        compiler_params=pltpu.CompilerParams(needs_layout_passes=False),
        out_type=jax.ShapeDtypeStruct((_BATCH, _DIM), jnp.float32),
        scratch_types=[
            pltpu.VMEM((bpw,), jnp.int32),
            pltpu.VMEM((_DIM, _NB * 128), jnp.float32),
            pltpu.VMEM((bpw, _DIM), jnp.float32),
            pltpu.SemaphoreType.DMA((_NB,)),
        ],
    )
    def gk(embt_hbm, idx_hbm, out_hbm, idx_v, blk_v, rows_v, sems):
        wid = lax.axis_index("s") * nc + lax.axis_index("c")
        base = wid * bpw
        pltpu.sync_copy(idx_hbm.at[pl.ds(base, bpw)], idx_v)

        def token_col(i):
            group = idx_v[pl.ds(i - i % nl, nl)]
            sel = lax.iota(jnp.int32, nl) == (i % nl)
            return lax.reduce_max(jnp.where(sel, group, 0), (0,))

        def fire(i):
            col = token_col(i)
            col0 = pl.multiple_of((col // 128) * 128, 128)
            s = i % _NB
            return (
                pltpu.async_copy(
                    embt_hbm.at[:, pl.ds(col0, 128)],
                    blk_v.at[:, pl.ds(s * 128, 128)],
                    sems.at[s],
                ),
                col - col0,
            )

        ring = [fire(i) for i in range(_NB)]
        for i in range(bpw):
            desc, r = ring[i % _NB]
            desc.wait()
            lane = i % _NB * 128 + r
            for k in range(_DIM // nl):
                d = lax.iota(jnp.int32, nl) + k * nl
                vals = plsc.load_gather(blk_v, [d, jnp.full((nl,), 0, jnp.int32) + lane])
                plsc.store_scatter(
                    rows_v, [jnp.full((nl,), i, jnp.int32), d], vals
                )
            if i + _NB < bpw:
                ring[i % _NB] = fire(i + _NB)
        pltpu.sync_copy(rows_v, out_hbm.at[pl.ds(base, bpw)])

    return gk(embt, idx)


def _matmul_body(x_ref, w_ref, b_ref, out_ref):
    lhs = jnp.concatenate([w_ref[...], b_ref[...]], axis=0).astype(jnp.bfloat16)
    rhs = jnp.concatenate(
        [x_ref[...], jnp.ones((_BATCH, 1), jnp.float32)], axis=1
    ).astype(jnp.bfloat16)
    out_ref[...] = jax.lax.dot_general(
        lhs, rhs, (((0,), (1,)), ((), ())), preferred_element_type=jnp.float32
    )


def kernel(input_ids, emb, W, b):
    idx = input_ids.astype(jnp.int32)
    embt = jnp.swapaxes(emb, 0, 1)
    x = _gather_rows_sc(embt, idx)
    b2 = b.reshape(1, _VOCAB)
    logits_t = pl.pallas_call(
        _matmul_body,
        grid=(pl.cdiv(_VOCAB, _TV),),
        in_specs=[
            pl.BlockSpec((_BATCH, _DIM), lambda i: (0, 0)),
            pl.BlockSpec((_DIM, _TV), lambda i: (0, i)),
            pl.BlockSpec((1, _TV), lambda i: (0, i)),
        ],
        out_specs=pl.BlockSpec((_TV, _BATCH), lambda i: (i, 0)),
        out_shape=jax.ShapeDtypeStruct((_VOCAB, _BATCH), jnp.float32),
        compiler_params=pltpu.CompilerParams(
            dimension_semantics=("arbitrary",),
        ),
    )(x, W, b2)
    return jnp.transpose(logits_t)
